# Initial kernel scaffold; baseline (speedup 1.0000x reference)
import jax
import jax.numpy as jnp
from jax.experimental import pallas as pl


def kernel(xyz1, xyz2, points1, points2, conv1_w, conv1_b, bn1_g, bn1_b, conv2_w, conv2_b, bn2_g, bn2_b):
    raise NotImplementedError("write your pallas kernel here")


# speedup vs baseline: 26.5949x; 26.5949x over previous
import jax
import jax.numpy as jnp
from jax.experimental import pallas as pl


def _stage1_body(x1_ref, x2_ref, p1_ref, p2_ref, w1a_ref, w1b_ref, b1_ref,
                 y1_ref, st_ref):
    a = x1_ref[0]
    c = x2_ref[0]
    mm = jnp.dot(c, a, preferred_element_type=jnp.float32)
    n1 = (a[0:1, :] * a[0:1, :] + a[1:2, :] * a[1:2, :]
          + a[2:3, :] * a[2:3, :])
    n2 = (c[:, 0:1] * c[:, 0:1] + c[:, 1:2] * c[:, 1:2]
          + c[:, 2:3] * c[:, 2:3])
    d = (-2.0 * mm + n1) + n2
    S = d.shape[0]
    row = jax.lax.broadcasted_iota(jnp.int32, d.shape, 0)
    big = jnp.float32(3.0e38)
    vals, onehots = [], []
    dd = d
    for _ in range(3):
        mn = jnp.min(dd, axis=0, keepdims=True)
        am = jnp.min(jnp.where(dd == mn, row, S), axis=0, keepdims=True)
        oh = row == am
        vals.append(mn)
        onehots.append(oh)
        dd = jnp.where(oh, big, dd)
    r0 = 1.0 / (vals[0] + 1e-8)
    r1 = 1.0 / (vals[1] + 1e-8)
    r2 = 1.0 / (vals[2] + 1e-8)
    norm = r0 + r1 + r2
    w = (jnp.where(onehots[0], r0 / norm, 0.0)
         + jnp.where(onehots[1], r1 / norm, 0.0)
         + jnp.where(onehots[2], r2 / norm, 0.0))
    interp = jnp.dot(p2_ref[0], w, preferred_element_type=jnp.float32)
    y = (jnp.dot(w1a_ref[...], p1_ref[0], preferred_element_type=jnp.float32)
         + jnp.dot(w1b_ref[...], interp, preferred_element_type=jnp.float32)
         + b1_ref[...])
    y1_ref[0] = y
    s = jnp.sum(y, axis=1, keepdims=True)
    ss = jnp.sum(y * y, axis=1, keepdims=True)
    st_ref[0] = jnp.concatenate(
        [s, ss, jnp.zeros((y.shape[0], 6), jnp.float32)], axis=1)


def _stage2_body(y1_ref, sc_ref, sh_ref, w2_ref, b2_ref, y2_ref, st_ref):
    y1n = jnp.maximum(y1_ref[0] * sc_ref[...] + sh_ref[...], 0.0)
    z = jnp.dot(w2_ref[...], y1n, preferred_element_type=jnp.float32) + b2_ref[...]
    y2_ref[0] = z
    s = jnp.sum(z, axis=1, keepdims=True)
    ss = jnp.sum(z * z, axis=1, keepdims=True)
    st_ref[0] = jnp.concatenate(
        [s, ss, jnp.zeros((z.shape[0], 6), jnp.float32)], axis=1)


def _stage3_body(y2_ref, sc_ref, sh_ref, out_ref):
    out_ref[0] = jnp.maximum(y2_ref[0] * sc_ref[...] + sh_ref[...], 0.0)


def kernel(xyz1, xyz2, points1, points2, conv1_w, conv1_b, bn1_g, bn1_b,
           conv2_w, conv2_b, bn2_g, bn2_b):
    f32 = jnp.float32
    B, _, N = xyz1.shape
    S = xyz2.shape[2]
    D1 = points1.shape[1]
    C1, Cin1 = conv1_w.shape
    C2 = conv2_w.shape[0]
    D2 = Cin1 - D1

    x2t = jnp.transpose(xyz2, (0, 2, 1))
    w1a = conv1_w[:, :D1]
    w1b = conv1_w[:, D1:]
    b1 = conv1_b[:, None]
    b2 = conv2_b[:, None]

    BN1 = 512
    nb1 = N // BN1
    y1, st1 = pl.pallas_call(
        _stage1_body,
        grid=(B, nb1),
        in_specs=[
            pl.BlockSpec((1, 3, BN1), lambda b, n: (b, 0, n)),
            pl.BlockSpec((1, S, 3), lambda b, n: (b, 0, 0)),
            pl.BlockSpec((1, D1, BN1), lambda b, n: (b, 0, n)),
            pl.BlockSpec((1, D2, S), lambda b, n: (b, 0, 0)),
            pl.BlockSpec((C1, D1), lambda b, n: (0, 0)),
            pl.BlockSpec((C1, D2), lambda b, n: (0, 0)),
            pl.BlockSpec((C1, 1), lambda b, n: (0, 0)),
        ],
        out_specs=[
            pl.BlockSpec((1, C1, BN1), lambda b, n: (b, 0, n)),
            pl.BlockSpec((1, C1, 8), lambda b, n: (b * nb1 + n, 0, 0)),
        ],
        out_shape=[
            jax.ShapeDtypeStruct((B, C1, N), f32),
            jax.ShapeDtypeStruct((B * nb1, C1, 8), f32),
        ],
    )(xyz1, x2t, points1, points2, w1a, w1b, b1)

    cnt = f32(B * N)
    s1 = jnp.sum(st1[:, :, 0], axis=0)[:, None]
    ss1 = jnp.sum(st1[:, :, 1], axis=0)[:, None]
    m1 = s1 / cnt
    v1 = ss1 / cnt - m1 * m1
    rstd1 = jax.lax.rsqrt(v1 + 1e-5)
    scale1 = bn1_g[:, None] * rstd1
    shift1 = bn1_b[:, None] - m1 * scale1

    BN2 = 2048
    nb2 = N // BN2
    y2, st2 = pl.pallas_call(
        _stage2_body,
        grid=(B, nb2),
        in_specs=[
            pl.BlockSpec((1, C1, BN2), lambda b, n: (b, 0, n)),
            pl.BlockSpec((C1, 1), lambda b, n: (0, 0)),
            pl.BlockSpec((C1, 1), lambda b, n: (0, 0)),
            pl.BlockSpec((C2, C1), lambda b, n: (0, 0)),
            pl.BlockSpec((C2, 1), lambda b, n: (0, 0)),
        ],
        out_specs=[
            pl.BlockSpec((1, C2, BN2), lambda b, n: (b, 0, n)),
            pl.BlockSpec((1, C2, 8), lambda b, n: (b * nb2 + n, 0, 0)),
        ],
        out_shape=[
            jax.ShapeDtypeStruct((B, C2, N), f32),
            jax.ShapeDtypeStruct((B * nb2, C2, 8), f32),
        ],
    )(y1, scale1, shift1, conv2_w, b2)

    s2 = jnp.sum(st2[:, :, 0], axis=0)[:, None]
    ss2 = jnp.sum(st2[:, :, 1], axis=0)[:, None]
    m2 = s2 / cnt
    v2 = ss2 / cnt - m2 * m2
    rstd2 = jax.lax.rsqrt(v2 + 1e-5)
    scale2 = bn2_g[:, None] * rstd2
    shift2 = bn2_b[:, None] - m2 * scale2

    BN3 = 4096
    out = pl.pallas_call(
        _stage3_body,
        grid=(B, N // BN3),
        in_specs=[
            pl.BlockSpec((1, C2, BN3), lambda b, n: (b, 0, n)),
            pl.BlockSpec((C2, 1), lambda b, n: (0, 0)),
            pl.BlockSpec((C2, 1), lambda b, n: (0, 0)),
        ],
        out_specs=pl.BlockSpec((1, C2, BN3), lambda b, n: (b, 0, n)),
        out_shape=jax.ShapeDtypeStruct((B, C2, N), f32),
    )(y2, scale2, shift2)

    return out


# speedup vs baseline: 29.9708x; 1.1269x over previous
import jax
import jax.numpy as jnp
from jax.experimental import pallas as pl


def _stage1_body(x1_ref, x2_ref, p1_ref, p2_ref, w1a_ref, w1b_ref, b1_ref,
                 y1_ref, st_ref):
    a = x1_ref[0]
    c = x2_ref[0]
    mm = jnp.dot(c, a, preferred_element_type=jnp.float32)
    n1 = (a[0:1, :] * a[0:1, :] + a[1:2, :] * a[1:2, :]
          + a[2:3, :] * a[2:3, :])
    n2 = (c[:, 0:1] * c[:, 0:1] + c[:, 1:2] * c[:, 1:2]
          + c[:, 2:3] * c[:, 2:3])
    d = (-2.0 * mm + n1) + n2
    S = d.shape[0]
    row = jax.lax.broadcasted_iota(jnp.int32, d.shape, 0)
    big = jnp.float32(3.0e38)
    vals, onehots = [], []
    dd = d
    for _ in range(3):
        mn = jnp.min(dd, axis=0, keepdims=True)
        am = jnp.min(jnp.where(dd == mn, row, S), axis=0, keepdims=True)
        oh = row == am
        vals.append(mn)
        onehots.append(oh)
        dd = jnp.where(oh, big, dd)
    r0 = 1.0 / (vals[0] + 1e-8)
    r1 = 1.0 / (vals[1] + 1e-8)
    r2 = 1.0 / (vals[2] + 1e-8)
    norm = r0 + r1 + r2
    w = (jnp.where(onehots[0], r0 / norm, 0.0)
         + jnp.where(onehots[1], r1 / norm, 0.0)
         + jnp.where(onehots[2], r2 / norm, 0.0))
    interp = jnp.dot(p2_ref[0], w, preferred_element_type=jnp.float32)
    y = (jnp.dot(w1a_ref[...], p1_ref[0], preferred_element_type=jnp.float32)
         + jnp.dot(w1b_ref[...], interp, preferred_element_type=jnp.float32)
         + b1_ref[...])
    y1_ref[0] = y
    s = jnp.sum(y, axis=1, keepdims=True)
    ss = jnp.sum(y * y, axis=1, keepdims=True)
    st_ref[0] = jnp.concatenate(
        [s, ss, jnp.zeros((y.shape[0], 6), jnp.float32)], axis=1)


def _stage2_body(y1_ref, sc_ref, sh_ref, w2_ref, b2_ref, y2_ref, st_ref):
    y1n = jnp.maximum(y1_ref[0] * sc_ref[...] + sh_ref[...], 0.0)
    z = jnp.dot(w2_ref[...], y1n, preferred_element_type=jnp.float32) + b2_ref[...]
    y2_ref[0] = z
    s = jnp.sum(z, axis=1, keepdims=True)
    ss = jnp.sum(z * z, axis=1, keepdims=True)
    st_ref[0] = jnp.concatenate(
        [s, ss, jnp.zeros((z.shape[0], 6), jnp.float32)], axis=1)


def _stage3_body(y2_ref, sc_ref, sh_ref, out_ref):
    out_ref[0] = jnp.maximum(y2_ref[0] * sc_ref[...] + sh_ref[...], 0.0)


def kernel(xyz1, xyz2, points1, points2, conv1_w, conv1_b, bn1_g, bn1_b,
           conv2_w, conv2_b, bn2_g, bn2_b):
    f32 = jnp.float32
    B, _, N = xyz1.shape
    S = xyz2.shape[2]
    D1 = points1.shape[1]
    C1, Cin1 = conv1_w.shape
    C2 = conv2_w.shape[0]
    D2 = Cin1 - D1

    x2t = jnp.transpose(xyz2, (0, 2, 1))
    w1a = conv1_w[:, :D1]
    w1b = conv1_w[:, D1:]
    b1 = conv1_b[:, None]
    b2 = conv2_b[:, None]

    BN1 = 1024
    nb1 = N // BN1
    y1, st1 = pl.pallas_call(
        _stage1_body,
        grid=(B, nb1),
        in_specs=[
            pl.BlockSpec((1, 3, BN1), lambda b, n: (b, 0, n)),
            pl.BlockSpec((1, S, 3), lambda b, n: (b, 0, 0)),
            pl.BlockSpec((1, D1, BN1), lambda b, n: (b, 0, n)),
            pl.BlockSpec((1, D2, S), lambda b, n: (b, 0, 0)),
            pl.BlockSpec((C1, D1), lambda b, n: (0, 0)),
            pl.BlockSpec((C1, D2), lambda b, n: (0, 0)),
            pl.BlockSpec((C1, 1), lambda b, n: (0, 0)),
        ],
        out_specs=[
            pl.BlockSpec((1, C1, BN1), lambda b, n: (b, 0, n)),
            pl.BlockSpec((1, C1, 8), lambda b, n: (b * nb1 + n, 0, 0)),
        ],
        out_shape=[
            jax.ShapeDtypeStruct((B, C1, N), f32),
            jax.ShapeDtypeStruct((B * nb1, C1, 8), f32),
        ],
    )(xyz1, x2t, points1, points2, w1a, w1b, b1)

    cnt = f32(B * N)
    s1 = jnp.sum(st1[:, :, 0], axis=0)[:, None]
    ss1 = jnp.sum(st1[:, :, 1], axis=0)[:, None]
    m1 = s1 / cnt
    v1 = ss1 / cnt - m1 * m1
    rstd1 = jax.lax.rsqrt(v1 + 1e-5)
    scale1 = bn1_g[:, None] * rstd1
    shift1 = bn1_b[:, None] - m1 * scale1

    BN2 = 2048
    nb2 = N // BN2
    y2, st2 = pl.pallas_call(
        _stage2_body,
        grid=(B, nb2),
        in_specs=[
            pl.BlockSpec((1, C1, BN2), lambda b, n: (b, 0, n)),
            pl.BlockSpec((C1, 1), lambda b, n: (0, 0)),
            pl.BlockSpec((C1, 1), lambda b, n: (0, 0)),
            pl.BlockSpec((C2, C1), lambda b, n: (0, 0)),
            pl.BlockSpec((C2, 1), lambda b, n: (0, 0)),
        ],
        out_specs=[
            pl.BlockSpec((1, C2, BN2), lambda b, n: (b, 0, n)),
            pl.BlockSpec((1, C2, 8), lambda b, n: (b * nb2 + n, 0, 0)),
        ],
        out_shape=[
            jax.ShapeDtypeStruct((B, C2, N), f32),
            jax.ShapeDtypeStruct((B * nb2, C2, 8), f32),
        ],
    )(y1, scale1, shift1, conv2_w, b2)

    s2 = jnp.sum(st2[:, :, 0], axis=0)[:, None]
    ss2 = jnp.sum(st2[:, :, 1], axis=0)[:, None]
    m2 = s2 / cnt
    v2 = ss2 / cnt - m2 * m2
    rstd2 = jax.lax.rsqrt(v2 + 1e-5)
    scale2 = bn2_g[:, None] * rstd2
    shift2 = bn2_b[:, None] - m2 * scale2

    BN3 = 4096
    out = pl.pallas_call(
        _stage3_body,
        grid=(B, N // BN3),
        in_specs=[
            pl.BlockSpec((1, C2, BN3), lambda b, n: (b, 0, n)),
            pl.BlockSpec((C2, 1), lambda b, n: (0, 0)),
            pl.BlockSpec((C2, 1), lambda b, n: (0, 0)),
        ],
        out_specs=pl.BlockSpec((1, C2, BN3), lambda b, n: (b, 0, n)),
        out_shape=jax.ShapeDtypeStruct((B, C2, N), f32),
    )(y2, scale2, shift2)

    return out


# speedup vs baseline: 32.8173x; 1.0950x over previous
import jax
import jax.numpy as jnp
from jax.experimental import pallas as pl


def _stage1_body(x1_ref, x2_ref, p1_ref, p2_ref, w1a_ref, w1b_ref, b1_ref,
                 y1_ref, st_ref):
    a = x1_ref[0]
    c = x2_ref[0]
    mm = jnp.dot(c, a, preferred_element_type=jnp.float32)
    n1 = (a[0:1, :] * a[0:1, :] + a[1:2, :] * a[1:2, :]
          + a[2:3, :] * a[2:3, :])
    n2 = (c[:, 0:1] * c[:, 0:1] + c[:, 1:2] * c[:, 1:2]
          + c[:, 2:3] * c[:, 2:3])
    d = (-2.0 * mm + n1) + n2
    S = d.shape[0]
    row = jax.lax.broadcasted_iota(jnp.int32, d.shape, 0)
    big = jnp.float32(3.0e38)
    vals, onehots = [], []
    dd = d
    for _ in range(3):
        mn = jnp.min(dd, axis=0, keepdims=True)
        am = jnp.min(jnp.where(dd == mn, row, S), axis=0, keepdims=True)
        oh = row == am
        vals.append(mn)
        onehots.append(oh)
        dd = jnp.where(oh, big, dd)
    r0 = 1.0 / (vals[0] + 1e-8)
    r1 = 1.0 / (vals[1] + 1e-8)
    r2 = 1.0 / (vals[2] + 1e-8)
    norm = r0 + r1 + r2
    w = (jnp.where(onehots[0], r0 / norm, 0.0)
         + jnp.where(onehots[1], r1 / norm, 0.0)
         + jnp.where(onehots[2], r2 / norm, 0.0))
    interp = jnp.dot(p2_ref[0], w, preferred_element_type=jnp.float32)
    y = (jnp.dot(w1a_ref[...], p1_ref[0], preferred_element_type=jnp.float32)
         + jnp.dot(w1b_ref[...], interp, preferred_element_type=jnp.float32)
         + b1_ref[...])
    y1_ref[0] = y
    s = jnp.sum(y, axis=1, keepdims=True)
    ss = jnp.sum(y * y, axis=1, keepdims=True)
    st_ref[0] = jnp.concatenate(
        [s, ss, jnp.zeros((y.shape[0], 6), jnp.float32)], axis=1)


def _stage2_body(y1_ref, sc_ref, sh_ref, w2_ref, b2_ref, y2_ref, st_ref):
    y1n = jnp.maximum(y1_ref[0] * sc_ref[...] + sh_ref[...], 0.0)
    z = jnp.dot(w2_ref[...], y1n, preferred_element_type=jnp.float32) + b2_ref[...]
    y2_ref[0] = z
    s = jnp.sum(z, axis=1, keepdims=True)
    ss = jnp.sum(z * z, axis=1, keepdims=True)
    st_ref[0] = jnp.concatenate(
        [s, ss, jnp.zeros((z.shape[0], 6), jnp.float32)], axis=1)


def _stage3_body(y2_ref, sc_ref, sh_ref, out_ref):
    out_ref[0] = jnp.maximum(y2_ref[0] * sc_ref[...] + sh_ref[...], 0.0)


def kernel(xyz1, xyz2, points1, points2, conv1_w, conv1_b, bn1_g, bn1_b,
           conv2_w, conv2_b, bn2_g, bn2_b):
    f32 = jnp.float32
    B, _, N = xyz1.shape
    S = xyz2.shape[2]
    D1 = points1.shape[1]
    C1, Cin1 = conv1_w.shape
    C2 = conv2_w.shape[0]
    D2 = Cin1 - D1

    x2t = jnp.transpose(xyz2, (0, 2, 1))
    w1a = conv1_w[:, :D1]
    w1b = conv1_w[:, D1:]
    b1 = conv1_b[:, None]
    b2 = conv2_b[:, None]

    BN1 = 2048
    nb1 = N // BN1
    y1, st1 = pl.pallas_call(
        _stage1_body,
        grid=(B, nb1),
        in_specs=[
            pl.BlockSpec((1, 3, BN1), lambda b, n: (b, 0, n)),
            pl.BlockSpec((1, S, 3), lambda b, n: (b, 0, 0)),
            pl.BlockSpec((1, D1, BN1), lambda b, n: (b, 0, n)),
            pl.BlockSpec((1, D2, S), lambda b, n: (b, 0, 0)),
            pl.BlockSpec((C1, D1), lambda b, n: (0, 0)),
            pl.BlockSpec((C1, D2), lambda b, n: (0, 0)),
            pl.BlockSpec((C1, 1), lambda b, n: (0, 0)),
        ],
        out_specs=[
            pl.BlockSpec((1, C1, BN1), lambda b, n: (b, 0, n)),
            pl.BlockSpec((1, C1, 8), lambda b, n: (b * nb1 + n, 0, 0)),
        ],
        out_shape=[
            jax.ShapeDtypeStruct((B, C1, N), f32),
            jax.ShapeDtypeStruct((B * nb1, C1, 8), f32),
        ],
    )(xyz1, x2t, points1, points2, w1a, w1b, b1)

    cnt = f32(B * N)
    s1 = jnp.sum(st1[:, :, 0], axis=0)[:, None]
    ss1 = jnp.sum(st1[:, :, 1], axis=0)[:, None]
    m1 = s1 / cnt
    v1 = ss1 / cnt - m1 * m1
    rstd1 = jax.lax.rsqrt(v1 + 1e-5)
    scale1 = bn1_g[:, None] * rstd1
    shift1 = bn1_b[:, None] - m1 * scale1

    BN2 = 4096
    nb2 = N // BN2
    y2, st2 = pl.pallas_call(
        _stage2_body,
        grid=(B, nb2),
        in_specs=[
            pl.BlockSpec((1, C1, BN2), lambda b, n: (b, 0, n)),
            pl.BlockSpec((C1, 1), lambda b, n: (0, 0)),
            pl.BlockSpec((C1, 1), lambda b, n: (0, 0)),
            pl.BlockSpec((C2, C1), lambda b, n: (0, 0)),
            pl.BlockSpec((C2, 1), lambda b, n: (0, 0)),
        ],
        out_specs=[
            pl.BlockSpec((1, C2, BN2), lambda b, n: (b, 0, n)),
            pl.BlockSpec((1, C2, 8), lambda b, n: (b * nb2 + n, 0, 0)),
        ],
        out_shape=[
            jax.ShapeDtypeStruct((B, C2, N), f32),
            jax.ShapeDtypeStruct((B * nb2, C2, 8), f32),
        ],
    )(y1, scale1, shift1, conv2_w, b2)

    s2 = jnp.sum(st2[:, :, 0], axis=0)[:, None]
    ss2 = jnp.sum(st2[:, :, 1], axis=0)[:, None]
    m2 = s2 / cnt
    v2 = ss2 / cnt - m2 * m2
    rstd2 = jax.lax.rsqrt(v2 + 1e-5)
    scale2 = bn2_g[:, None] * rstd2
    shift2 = bn2_b[:, None] - m2 * scale2

    BN3 = 4096
    out = pl.pallas_call(
        _stage3_body,
        grid=(B, N // BN3),
        in_specs=[
            pl.BlockSpec((1, C2, BN3), lambda b, n: (b, 0, n)),
            pl.BlockSpec((C2, 1), lambda b, n: (0, 0)),
            pl.BlockSpec((C2, 1), lambda b, n: (0, 0)),
        ],
        out_specs=pl.BlockSpec((1, C2, BN3), lambda b, n: (b, 0, n)),
        out_shape=jax.ShapeDtypeStruct((B, C2, N), f32),
    )(y2, scale2, shift2)

    return out


# speedup vs baseline: 32.8461x; 1.0009x over previous
import jax
import jax.numpy as jnp
from jax.experimental import pallas as pl


def _stage1_body(x1_ref, x2_ref, p1_ref, p2_ref, w1a_ref, w1b_ref, b1_ref,
                 y1_ref, st_ref):
    a = x1_ref[0]
    c = x2_ref[0]
    mm = jnp.dot(c, a, preferred_element_type=jnp.float32)
    n1 = (a[0:1, :] * a[0:1, :] + a[1:2, :] * a[1:2, :]
          + a[2:3, :] * a[2:3, :])
    n2 = (c[:, 0:1] * c[:, 0:1] + c[:, 1:2] * c[:, 1:2]
          + c[:, 2:3] * c[:, 2:3])
    d = (-2.0 * mm + n1) + n2
    S = d.shape[0]
    row = jax.lax.broadcasted_iota(jnp.int32, d.shape, 0)
    big = jnp.float32(3.0e38)
    vals, onehots = [], []
    dd = d
    for _ in range(3):
        mn = jnp.min(dd, axis=0, keepdims=True)
        am = jnp.min(jnp.where(dd == mn, row, S), axis=0, keepdims=True)
        oh = row == am
        vals.append(mn)
        onehots.append(oh)
        dd = jnp.where(oh, big, dd)
    r0 = 1.0 / (vals[0] + 1e-8)
    r1 = 1.0 / (vals[1] + 1e-8)
    r2 = 1.0 / (vals[2] + 1e-8)
    norm = r0 + r1 + r2
    w = (jnp.where(onehots[0], r0 / norm, 0.0)
         + jnp.where(onehots[1], r1 / norm, 0.0)
         + jnp.where(onehots[2], r2 / norm, 0.0))
    interp = jnp.dot(p2_ref[0], w, preferred_element_type=jnp.float32)
    y = (jnp.dot(w1a_ref[...], p1_ref[0], preferred_element_type=jnp.float32)
         + jnp.dot(w1b_ref[...], interp, preferred_element_type=jnp.float32)
         + b1_ref[...])
    y1_ref[0] = y
    s = jnp.sum(y, axis=1, keepdims=True)
    ss = jnp.sum(y * y, axis=1, keepdims=True)
    st_ref[0] = jnp.concatenate(
        [s, ss, jnp.zeros((y.shape[0], 6), jnp.float32)], axis=1)


def _stage2_body(y1_ref, sc_ref, sh_ref, w2_ref, b2_ref, y2_ref, st_ref):
    y1n = jnp.maximum(y1_ref[0] * sc_ref[...] + sh_ref[...], 0.0)
    z = jnp.dot(w2_ref[...], y1n, preferred_element_type=jnp.float32) + b2_ref[...]
    y2_ref[0] = z
    s = jnp.sum(z, axis=1, keepdims=True)
    ss = jnp.sum(z * z, axis=1, keepdims=True)
    st_ref[0] = jnp.concatenate(
        [s, ss, jnp.zeros((z.shape[0], 6), jnp.float32)], axis=1)


def _stage3_body(y2_ref, sc_ref, sh_ref, out_ref):
    out_ref[0] = jnp.maximum(y2_ref[0] * sc_ref[...] + sh_ref[...], 0.0)


def kernel(xyz1, xyz2, points1, points2, conv1_w, conv1_b, bn1_g, bn1_b,
           conv2_w, conv2_b, bn2_g, bn2_b):
    f32 = jnp.float32
    B, _, N = xyz1.shape
    S = xyz2.shape[2]
    D1 = points1.shape[1]
    C1, Cin1 = conv1_w.shape
    C2 = conv2_w.shape[0]
    D2 = Cin1 - D1

    x2t = jnp.transpose(xyz2, (0, 2, 1))
    w1a = conv1_w[:, :D1]
    w1b = conv1_w[:, D1:]
    b1 = conv1_b[:, None]
    b2 = conv2_b[:, None]

    BN1 = 4096
    nb1 = N // BN1
    y1, st1 = pl.pallas_call(
        _stage1_body,
        grid=(B, nb1),
        in_specs=[
            pl.BlockSpec((1, 3, BN1), lambda b, n: (b, 0, n)),
            pl.BlockSpec((1, S, 3), lambda b, n: (b, 0, 0)),
            pl.BlockSpec((1, D1, BN1), lambda b, n: (b, 0, n)),
            pl.BlockSpec((1, D2, S), lambda b, n: (b, 0, 0)),
            pl.BlockSpec((C1, D1), lambda b, n: (0, 0)),
            pl.BlockSpec((C1, D2), lambda b, n: (0, 0)),
            pl.BlockSpec((C1, 1), lambda b, n: (0, 0)),
        ],
        out_specs=[
            pl.BlockSpec((1, C1, BN1), lambda b, n: (b, 0, n)),
            pl.BlockSpec((1, C1, 8), lambda b, n: (b * nb1 + n, 0, 0)),
        ],
        out_shape=[
            jax.ShapeDtypeStruct((B, C1, N), f32),
            jax.ShapeDtypeStruct((B * nb1, C1, 8), f32),
        ],
    )(xyz1, x2t, points1, points2, w1a, w1b, b1)

    cnt = f32(B * N)
    s1 = jnp.sum(st1[:, :, 0], axis=0)[:, None]
    ss1 = jnp.sum(st1[:, :, 1], axis=0)[:, None]
    m1 = s1 / cnt
    v1 = ss1 / cnt - m1 * m1
    rstd1 = jax.lax.rsqrt(v1 + 1e-5)
    scale1 = bn1_g[:, None] * rstd1
    shift1 = bn1_b[:, None] - m1 * scale1

    BN2 = 4096
    nb2 = N // BN2
    y2, st2 = pl.pallas_call(
        _stage2_body,
        grid=(B, nb2),
        in_specs=[
            pl.BlockSpec((1, C1, BN2), lambda b, n: (b, 0, n)),
            pl.BlockSpec((C1, 1), lambda b, n: (0, 0)),
            pl.BlockSpec((C1, 1), lambda b, n: (0, 0)),
            pl.BlockSpec((C2, C1), lambda b, n: (0, 0)),
            pl.BlockSpec((C2, 1), lambda b, n: (0, 0)),
        ],
        out_specs=[
            pl.BlockSpec((1, C2, BN2), lambda b, n: (b, 0, n)),
            pl.BlockSpec((1, C2, 8), lambda b, n: (b * nb2 + n, 0, 0)),
        ],
        out_shape=[
            jax.ShapeDtypeStruct((B, C2, N), f32),
            jax.ShapeDtypeStruct((B * nb2, C2, 8), f32),
        ],
    )(y1, scale1, shift1, conv2_w, b2)

    s2 = jnp.sum(st2[:, :, 0], axis=0)[:, None]
    ss2 = jnp.sum(st2[:, :, 1], axis=0)[:, None]
    m2 = s2 / cnt
    v2 = ss2 / cnt - m2 * m2
    rstd2 = jax.lax.rsqrt(v2 + 1e-5)
    scale2 = bn2_g[:, None] * rstd2
    shift2 = bn2_b[:, None] - m2 * scale2

    BN3 = 4096
    out = pl.pallas_call(
        _stage3_body,
        grid=(B, N // BN3),
        in_specs=[
            pl.BlockSpec((1, C2, BN3), lambda b, n: (b, 0, n)),
            pl.BlockSpec((C2, 1), lambda b, n: (0, 0)),
            pl.BlockSpec((C2, 1), lambda b, n: (0, 0)),
        ],
        out_specs=pl.BlockSpec((1, C2, BN3), lambda b, n: (b, 0, n)),
        out_shape=jax.ShapeDtypeStruct((B, C2, N), f32),
    )(y2, scale2, shift2)

    return out
